# Initial kernel scaffold; baseline (speedup 1.0000x reference)
import jax
import jax.numpy as jnp
from jax.experimental import pallas as pl


def kernel(q, k, v):
    raise NotImplementedError("write your pallas kernel here")


# speedup vs baseline: 1.7295x; 1.7295x over previous
import functools

import jax
import jax.numpy as jnp
from jax.experimental import pallas as pl
from jax.experimental.pallas import tpu as pltpu

H = 16
HKV = 4
D = 64
SCALE = 0.125
B = 16
S = 256
REP = H // HKV
T = B * S


def _attn_kernel(q_ref, k_ref, v_ref, o_ref):
    q = q_ref[0, 0]
    k = k_ref[0, 0]
    v = v_ref[0, 0]
    logits = jax.lax.dot_general(
        q, k, (((1,), (1,)), ((), ())),
        preferred_element_type=jnp.float32) * SCALE
    row = jax.lax.broadcasted_iota(jnp.int32, (REP * S, S), 0) % S
    col = jax.lax.broadcasted_iota(jnp.int32, (REP * S, S), 1)
    logits = jnp.where(row >= col, logits, -1e30)
    m = jnp.max(logits, axis=1, keepdims=True)
    e = jnp.exp(logits - m)
    p = e / jnp.sum(e, axis=1, keepdims=True)
    o_ref[0, 0] = jax.lax.dot_general(
        p, v, (((1,), (0,)), ((), ())),
        preferred_element_type=jnp.float32)


@jax.jit
def kernel(q, k, v):
    qg = q.reshape(B, S, H, D).transpose(0, 2, 1, 3).reshape(B, HKV, REP * S, D)
    kg = k.reshape(B, S, HKV, D).transpose(0, 2, 1, 3)
    vg = v.reshape(B, S, HKV, D).transpose(0, 2, 1, 3)
    out = pl.pallas_call(
        _attn_kernel,
        grid=(B, HKV),
        in_specs=[
            pl.BlockSpec((1, 1, REP * S, D), lambda b, g: (b, g, 0, 0)),
            pl.BlockSpec((1, 1, S, D), lambda b, g: (b, g, 0, 0)),
            pl.BlockSpec((1, 1, S, D), lambda b, g: (b, g, 0, 0)),
        ],
        out_specs=pl.BlockSpec((1, 1, REP * S, D), lambda b, g: (b, g, 0, 0)),
        out_shape=jax.ShapeDtypeStruct((B, HKV, REP * S, D), jnp.float32),
        compiler_params=pltpu.CompilerParams(
            dimension_semantics=("parallel", "parallel")),
    )(qg, kg, vg)
    o = out.reshape(B, H, S, D).transpose(0, 2, 1, 3).reshape(T, H * D)
    return o


# speedup vs baseline: 3.9732x; 2.2973x over previous
import jax
import jax.numpy as jnp
from jax.experimental import pallas as pl
from jax.experimental.pallas import tpu as pltpu

H = 16
HKV = 4
D = 64
SCALE = 0.125
B = 16
S = 256
REP = H // HKV
T = B * S


def _attn_kernel(q_ref, k_ref, v_ref, o_ref):
    row = jax.lax.broadcasted_iota(jnp.int32, (S, S), 0)
    col = jax.lax.broadcasted_iota(jnp.int32, (S, S), 1)
    causal = row >= col
    for g in range(HKV):
        k = k_ref[:, g * D:(g + 1) * D]
        v = v_ref[:, g * D:(g + 1) * D]
        for r in range(REP):
            h = g * REP + r
            qr = q_ref[:, h * D:(h + 1) * D]
            logits = jax.lax.dot_general(
                qr, k, (((1,), (1,)), ((), ())),
                preferred_element_type=jnp.float32) * SCALE
            logits = jnp.where(causal, logits, -1e30)
            m = jnp.max(logits, axis=1, keepdims=True)
            e = jnp.exp(logits - m)
            p = e / jnp.sum(e, axis=1, keepdims=True)
            o_ref[:, h * D:(h + 1) * D] = jax.lax.dot_general(
                p, v, (((1,), (0,)), ((), ())),
                preferred_element_type=jnp.float32)


@jax.jit
def kernel(q, k, v):
    return pl.pallas_call(
        _attn_kernel,
        grid=(B,),
        in_specs=[
            pl.BlockSpec((S, H * D), lambda b: (b, 0)),
            pl.BlockSpec((S, HKV * D), lambda b: (b, 0)),
            pl.BlockSpec((S, HKV * D), lambda b: (b, 0)),
        ],
        out_specs=pl.BlockSpec((S, H * D), lambda b: (b, 0)),
        out_shape=jax.ShapeDtypeStruct((T, H * D), jnp.float32),
        compiler_params=pltpu.CompilerParams(
            dimension_semantics=("parallel",)),
    )(q, k, v)


# speedup vs baseline: 4.8027x; 1.2088x over previous
import jax
import jax.numpy as jnp
from jax.experimental import pallas as pl
from jax.experimental.pallas import tpu as pltpu

H = 16
HKV = 4
D = 64
SCALE = 0.125
B = 16
S = 256
REP = H // HKV
T = B * S


def _attn_kernel(q_ref, k_ref, v_ref, o_ref):
    row = jax.lax.broadcasted_iota(jnp.int32, (S, S), 0)
    col = jax.lax.broadcasted_iota(jnp.int32, (S, S), 1)
    causal = row >= col
    for g in range(HKV):
        k = k_ref[:, g * D:(g + 1) * D].astype(jnp.bfloat16)
        v = v_ref[:, g * D:(g + 1) * D].astype(jnp.bfloat16)
        for r in range(REP):
            h = g * REP + r
            qr = q_ref[:, h * D:(h + 1) * D].astype(jnp.bfloat16)
            logits = jax.lax.dot_general(
                qr, k, (((1,), (1,)), ((), ())),
                preferred_element_type=jnp.float32) * SCALE
            logits = jnp.where(causal, logits, -1e30)
            m = jnp.max(logits, axis=1, keepdims=True)
            e = jnp.exp(logits - m)
            p = (e / jnp.sum(e, axis=1, keepdims=True)).astype(jnp.bfloat16)
            o_ref[:, h * D:(h + 1) * D] = jax.lax.dot_general(
                p, v, (((1,), (0,)), ((), ())),
                preferred_element_type=jnp.float32)


@jax.jit
def kernel(q, k, v):
    return pl.pallas_call(
        _attn_kernel,
        grid=(B,),
        in_specs=[
            pl.BlockSpec((S, H * D), lambda b: (b, 0)),
            pl.BlockSpec((S, HKV * D), lambda b: (b, 0)),
            pl.BlockSpec((S, HKV * D), lambda b: (b, 0)),
        ],
        out_specs=pl.BlockSpec((S, H * D), lambda b: (b, 0)),
        out_shape=jax.ShapeDtypeStruct((T, H * D), jnp.float32),
        compiler_params=pltpu.CompilerParams(
            dimension_semantics=("parallel",)),
    )(q, k, v)
